# baseline (speedup 1.0000x reference)
import functools

import jax
import jax.numpy as jnp
from jax import lax
from jax.experimental import pallas as pl
from jax.experimental.pallas import tpu as pltpu
from jax.experimental.pallas import tpu_sc as plsc

_N, _E = 10000, 320000
_GD = 32
_EV = 20480
_E2 = _EV * 16
_NC, _NS = 2, 16
_EGA = 4
_CGA = 8
_RPA = _EV // _EGA
_RPD = _EV // 32
_CH = 512

_mesh = plsc.VectorSubcoreMesh(
    core_axis_name="c", subcore_axis_name="s", num_cores=_NC, num_subcores=_NS
)


_SLB = 4 * _N
_ECH = _CH * 16
_EPA = _E2 // _EGA
_EPD = _E2 // 32


@functools.partial(
    pl.kernel,
    out_type=jax.ShapeDtypeStruct((_EGA * _CGA * _SLB,), jnp.float32),
    mesh=_mesh,
    scratch_types=[
        [pltpu.VMEM((_N,), jnp.float32)] * 4,
        [pltpu.VMEM((_N,), jnp.float32)] * 4,
        pltpu.VMEM((_ECH,), jnp.int32),
        pltpu.VMEM((_ECH,), jnp.int32),
    ],
    compiler_params=pltpu.CompilerParams(needs_layout_passes=False),
)
def _sc_aggregate_body(gt_hbm, src_hbm, dst_hbm, zeros_hbm, out_hbm,
                       tbl_v, acc_v, src_v, dst_v):
    c = lax.axis_index("c")
    s = lax.axis_index("s")
    wid = s * _NC + c
    eg = wid // _CGA
    cg = wid % _CGA
    for t in range(4):
        pltpu.sync_copy(gt_hbm.at[pl.ds(cg * _SLB + t * _N, _N)], tbl_v[t])
        pltpu.sync_copy(zeros_hbm, acc_v[t])
    ebase = eg * _EPA

    @pl.loop(0, _EPA // _ECH)
    def _chunks(ch):
        pltpu.sync_copy(src_hbm.at[pl.ds(ebase + ch * _ECH, _ECH)], src_v)
        pltpu.sync_copy(dst_hbm.at[pl.ds(ebase + ch * _ECH, _ECH)], dst_v)

        @pl.loop(0, _CH, unroll=8)
        def _rows(r):
            srcv = src_v[pl.ds(r * 16, 16)]
            dstv = dst_v[pl.ds(r * 16, 16)]
            m = srcv >= 0
            srcc = jnp.maximum(srcv, 0)
            dstc = jnp.maximum(dstv, 0)
            for t in range(4):
                v = plsc.load_gather(tbl_v[t], [srcc], mask=m)
                plsc.addupdate_scatter(acc_v[t], [dstc], v, mask=m)

    for t in range(4):
        pltpu.sync_copy(acc_v[t], out_hbm.at[pl.ds(wid * _SLB + t * _N, _N)])


@functools.partial(
    pl.kernel,
    out_type=jax.ShapeDtypeStruct((32 * _N,), jnp.float32),
    mesh=_mesh,
    scratch_types=[
        pltpu.VMEM((_N,), jnp.float32),
        pltpu.VMEM((_EPD,), jnp.int32),
    ],
    compiler_params=pltpu.CompilerParams(needs_layout_passes=False),
)
def _sc_degree_body(dst_hbm, zeros_hbm, out_hbm, acc_v, dst_v):
    c = lax.axis_index("c")
    s = lax.axis_index("s")
    wid = s * _NC + c
    pltpu.sync_copy(zeros_hbm, acc_v)
    pltpu.sync_copy(dst_hbm.at[pl.ds(wid * _EPD, _EPD)], dst_v)

    @pl.loop(0, _EPD // 16, unroll=4)
    def _rows(r):
        dstv = dst_v[pl.ds(r * 16, 16)]
        m = dstv >= 0
        dstc = jnp.maximum(dstv, 0)
        one16 = jnp.full((16,), 1.0, jnp.float32)
        plsc.addupdate_scatter(acc_v, [dstc], one16, mask=m)

    pltpu.sync_copy(acc_v, out_hbm.at[pl.ds(wid * _N, _N)])


def _embed_body(x_ref, s_ref, r_ref, we1_ref, be1_ref, we2_ref, be2_ref,
                wg1_ref, z1_ref):
    sub = jnp.dot(s_ref[...], r_ref[...], preferred_element_type=jnp.float32)
    h = jnp.concatenate([x_ref[...], sub], axis=1)
    t = jnp.tanh(jnp.dot(h, we1_ref[...], preferred_element_type=jnp.float32)
                 + be1_ref[...])
    h2 = jnp.tanh(jnp.dot(t, we2_ref[...], preferred_element_type=jnp.float32)
                  + be2_ref[...])
    z1_ref[...] = jnp.dot(h2, wg1_ref[...],
                          preferred_element_type=jnp.float32).T


_embed_call = pl.pallas_call(
    _embed_body,
    out_shape=jax.ShapeDtypeStruct((_GD, _N), jnp.float32),
)


def _scale_body(d_ref, z1_ref, dinv_ref, g_ref):
    deg = jnp.sum(d_ref[...], axis=0, keepdims=True) + 1.0
    dinvT = lax.rsqrt(deg)
    dinv_ref[...] = dinvT
    g_ref[...] = dinvT * z1_ref[...]


_scale_call = pl.pallas_call(
    _scale_body,
    out_shape=[
        jax.ShapeDtypeStruct((1, _N), jnp.float32),
        jax.ShapeDtypeStruct((_GD, _N), jnp.float32),
    ],
)


def _agg_from_parts(p_ref):
    parts = []
    for b in range(_CGA):
        parts.append(p_ref[0, b] + p_ref[1, b] + p_ref[2, b] + p_ref[3, b])
    return jnp.concatenate(parts, axis=0)


def _layer_body(p_ref, g_ref, dinv_ref, b_ref, w_ref, gout_ref):
    dinvT = dinv_ref[...]
    aggT = _agg_from_parts(p_ref)
    hT = jnp.maximum(dinvT * (aggT + g_ref[...]) + b_ref[...], 0.0)
    gout_ref[...] = dinvT * jnp.dot(w_ref[...].T, hT,
                                    preferred_element_type=jnp.float32)


_layer_call = pl.pallas_call(
    _layer_body,
    out_shape=jax.ShapeDtypeStruct((_GD, _N), jnp.float32),
)


def _final_body(p_ref, g_ref, dinv_ref, b3_ref, wp1_ref, bp1_ref, wp2_ref,
                bp2_ref, pri_ref, out_ref):
    dinvT = dinv_ref[...]
    aggT = _agg_from_parts(p_ref)
    hT = jnp.maximum(dinvT * (aggT + g_ref[...]) + b3_ref[...], 0.0)
    h = hT.T
    t = jnp.tanh(jnp.dot(h, wp1_ref[...], preferred_element_type=jnp.float32)
                 + bp1_ref[...])
    o = jnp.tanh(jnp.dot(t, wp2_ref[...], preferred_element_type=jnp.float32)
                 + bp2_ref[...])
    out_ref[...] = o + pri_ref[...]


_final_call = pl.pallas_call(
    _final_body,
    out_shape=jax.ShapeDtypeStruct((_N, 16), jnp.float32),
)


def kernel(x, edge_index, priors, S, R, We1, be1, We2, be2,
           Wg1, bg1, Wg2, bg2, Wg3, bg3, Wp1, bp1, Wp2, bp2):
    f32, i32 = jnp.float32, jnp.int32
    pad = _E2 - _E
    srcp = jnp.concatenate([edge_index[0], jnp.full((pad,), -1, i32)])
    dstp = jnp.concatenate([edge_index[1], jnp.full((pad,), -1, i32)])
    zeros1 = jnp.zeros((_N,), f32)

    degp = _sc_degree_body(dstp, zeros1).reshape(32, _N)
    z1T = _embed_call(x, S, R, We1, be1.reshape(1, -1), We2,
                      be2.reshape(1, -1), Wg1)
    dinvT, g1T = _scale_call(degp, z1T)

    def _agg(gT):
        p = _sc_aggregate_body(gT.reshape(-1), srcp, dstp, zeros1)
        return p.reshape(_EGA, _CGA, 4, _N)

    p = _agg(g1T)
    g2T = _layer_call(p, g1T, dinvT, bg1.reshape(-1, 1), Wg2)
    p = _agg(g2T)
    g3T = _layer_call(p, g2T, dinvT, bg2.reshape(-1, 1), Wg3)
    p = _agg(g3T)
    out = _final_call(p, g3T, dinvT, bg3.reshape(-1, 1), Wp1,
                      bp1.reshape(1, -1), Wp2, bp2.reshape(1, -1), priors)
    return out


# speedup vs baseline: 1.0677x; 1.0677x over previous
import functools

import jax
import jax.numpy as jnp
from jax import lax
from jax.experimental import pallas as pl
from jax.experimental.pallas import tpu as pltpu
from jax.experimental.pallas import tpu_sc as plsc

_N, _E = 10000, 320000
_GD = 32
_EV = 20480
_E2 = _EV * 16
_NC, _NS = 2, 16
_EGA = 4
_CGA = 8
_RPA = _EV // _EGA
_RPD = _EV // 32
_CH = 512

_mesh = plsc.VectorSubcoreMesh(
    core_axis_name="c", subcore_axis_name="s", num_cores=_NC, num_subcores=_NS
)


_SLB = 4 * _N
_ECH = _CH * 16
_EPA = _E2 // _EGA
_EPD = _E2 // 32


@functools.partial(
    pl.kernel,
    out_type=jax.ShapeDtypeStruct((_EGA * _CGA * _SLB,), jnp.float32),
    mesh=_mesh,
    scratch_types=[
        [pltpu.VMEM((_N,), jnp.float32)] * 4,
        [pltpu.VMEM((_N,), jnp.float32)] * 4,
        pltpu.VMEM((_ECH,), jnp.int32),
        pltpu.VMEM((_ECH,), jnp.int32),
    ],
    compiler_params=pltpu.CompilerParams(needs_layout_passes=False),
)
def _sc_aggregate_body(gt_hbm, src_hbm, dst_hbm, zeros_hbm, out_hbm,
                       tbl_v, acc_v, src_v, dst_v):
    c = lax.axis_index("c")
    s = lax.axis_index("s")
    wid = s * _NC + c
    eg = wid // _CGA
    cg = wid % _CGA
    for t in range(4):
        pltpu.sync_copy(gt_hbm.at[pl.ds(cg * _SLB + t * _N, _N)], tbl_v[t])
        pltpu.sync_copy(zeros_hbm, acc_v[t])
    ebase = eg * _EPA

    zf = jnp.zeros((16,), jnp.float32)
    zi = jnp.zeros((16,), jnp.int32)

    @pl.loop(0, _EPA // _ECH, init_carry=(zf, zf, zf, zf, zi))
    def _chunks(ch, carry):
        pltpu.sync_copy(src_hbm.at[pl.ds(ebase + ch * _ECH, _ECH)], src_v)
        pltpu.sync_copy(dst_hbm.at[pl.ds(ebase + ch * _ECH, _ECH)], dst_v)

        @pl.loop(0, _CH, init_carry=carry, unroll=4)
        def _rows(r, cy):
            pv0, pv1, pv2, pv3, pdst = cy
            for t, pv in enumerate((pv0, pv1, pv2, pv3)):
                plsc.addupdate_scatter(acc_v[t], [pdst], pv)
            srcv = src_v[pl.ds(r * 16, 16)]
            dstv = dst_v[pl.ds(r * 16, 16)]
            m = srcv >= 0
            srcc = jnp.maximum(srcv, 0)
            dstc = jnp.maximum(dstv, 0)
            nv = []
            for t in range(4):
                g = plsc.load_gather(tbl_v[t], [srcc])
                nv.append(jnp.where(m, g, 0.0))
            return (nv[0], nv[1], nv[2], nv[3], dstc)

        return _rows

    fv0, fv1, fv2, fv3, fdst = _chunks
    for t, fv in enumerate((fv0, fv1, fv2, fv3)):
        plsc.addupdate_scatter(acc_v[t], [fdst], fv)

    for t in range(4):
        pltpu.sync_copy(acc_v[t], out_hbm.at[pl.ds(wid * _SLB + t * _N, _N)])


@functools.partial(
    pl.kernel,
    out_type=jax.ShapeDtypeStruct((32 * _N,), jnp.float32),
    mesh=_mesh,
    scratch_types=[
        pltpu.VMEM((_N,), jnp.float32),
        pltpu.VMEM((_EPD,), jnp.int32),
    ],
    compiler_params=pltpu.CompilerParams(needs_layout_passes=False),
)
def _sc_degree_body(dst_hbm, zeros_hbm, out_hbm, acc_v, dst_v):
    c = lax.axis_index("c")
    s = lax.axis_index("s")
    wid = s * _NC + c
    pltpu.sync_copy(zeros_hbm, acc_v)
    pltpu.sync_copy(dst_hbm.at[pl.ds(wid * _EPD, _EPD)], dst_v)

    @pl.loop(0, _EPD // 16, unroll=4)
    def _rows(r):
        dstv = dst_v[pl.ds(r * 16, 16)]
        m = dstv >= 0
        dstc = jnp.maximum(dstv, 0)
        one16 = jnp.full((16,), 1.0, jnp.float32)
        plsc.addupdate_scatter(acc_v, [dstc], one16, mask=m)

    pltpu.sync_copy(acc_v, out_hbm.at[pl.ds(wid * _N, _N)])


def _embed_body(x_ref, s_ref, r_ref, we1_ref, be1_ref, we2_ref, be2_ref,
                wg1_ref, z1_ref):
    sub = jnp.dot(s_ref[...], r_ref[...], preferred_element_type=jnp.float32)
    h = jnp.concatenate([x_ref[...], sub], axis=1)
    t = jnp.tanh(jnp.dot(h, we1_ref[...], preferred_element_type=jnp.float32)
                 + be1_ref[...])
    h2 = jnp.tanh(jnp.dot(t, we2_ref[...], preferred_element_type=jnp.float32)
                  + be2_ref[...])
    z1_ref[...] = jnp.dot(h2, wg1_ref[...],
                          preferred_element_type=jnp.float32).T


_embed_call = pl.pallas_call(
    _embed_body,
    out_shape=jax.ShapeDtypeStruct((_GD, _N), jnp.float32),
)


def _scale_body(d_ref, z1_ref, dinv_ref, g_ref):
    deg = jnp.sum(d_ref[...], axis=0, keepdims=True) + 1.0
    dinvT = lax.rsqrt(deg)
    dinv_ref[...] = dinvT
    g_ref[...] = dinvT * z1_ref[...]


_scale_call = pl.pallas_call(
    _scale_body,
    out_shape=[
        jax.ShapeDtypeStruct((1, _N), jnp.float32),
        jax.ShapeDtypeStruct((_GD, _N), jnp.float32),
    ],
)


def _agg_from_parts(p_ref):
    parts = []
    for b in range(_CGA):
        parts.append(p_ref[0, b] + p_ref[1, b] + p_ref[2, b] + p_ref[3, b])
    return jnp.concatenate(parts, axis=0)


def _layer_body(p_ref, g_ref, dinv_ref, b_ref, w_ref, gout_ref):
    dinvT = dinv_ref[...]
    aggT = _agg_from_parts(p_ref)
    hT = jnp.maximum(dinvT * (aggT + g_ref[...]) + b_ref[...], 0.0)
    gout_ref[...] = dinvT * jnp.dot(w_ref[...].T, hT,
                                    preferred_element_type=jnp.float32)


_layer_call = pl.pallas_call(
    _layer_body,
    out_shape=jax.ShapeDtypeStruct((_GD, _N), jnp.float32),
)


def _final_body(p_ref, g_ref, dinv_ref, b3_ref, wp1_ref, bp1_ref, wp2_ref,
                bp2_ref, pri_ref, out_ref):
    dinvT = dinv_ref[...]
    aggT = _agg_from_parts(p_ref)
    hT = jnp.maximum(dinvT * (aggT + g_ref[...]) + b3_ref[...], 0.0)
    h = hT.T
    t = jnp.tanh(jnp.dot(h, wp1_ref[...], preferred_element_type=jnp.float32)
                 + bp1_ref[...])
    o = jnp.tanh(jnp.dot(t, wp2_ref[...], preferred_element_type=jnp.float32)
                 + bp2_ref[...])
    out_ref[...] = o + pri_ref[...]


_final_call = pl.pallas_call(
    _final_body,
    out_shape=jax.ShapeDtypeStruct((_N, 16), jnp.float32),
)


def kernel(x, edge_index, priors, S, R, We1, be1, We2, be2,
           Wg1, bg1, Wg2, bg2, Wg3, bg3, Wp1, bp1, Wp2, bp2):
    f32, i32 = jnp.float32, jnp.int32
    pad = _E2 - _E
    srcp = jnp.concatenate([edge_index[0], jnp.full((pad,), -1, i32)])
    dstp = jnp.concatenate([edge_index[1], jnp.full((pad,), -1, i32)])
    zeros1 = jnp.zeros((_N,), f32)

    degp = _sc_degree_body(dstp, zeros1).reshape(32, _N)
    z1T = _embed_call(x, S, R, We1, be1.reshape(1, -1), We2,
                      be2.reshape(1, -1), Wg1)
    dinvT, g1T = _scale_call(degp, z1T)

    def _agg(gT):
        p = _sc_aggregate_body(gT.reshape(-1), srcp, dstp, zeros1)
        return p.reshape(_EGA, _CGA, 4, _N)

    p = _agg(g1T)
    g2T = _layer_call(p, g1T, dinvT, bg1.reshape(-1, 1), Wg2)
    p = _agg(g2T)
    g3T = _layer_call(p, g2T, dinvT, bg2.reshape(-1, 1), Wg3)
    p = _agg(g3T)
    out = _final_call(p, g3T, dinvT, bg3.reshape(-1, 1), Wp1,
                      bp1.reshape(1, -1), Wp2, bp2.reshape(1, -1), priors)
    return out


# speedup vs baseline: 1.0791x; 1.0107x over previous
import functools

import jax
import jax.numpy as jnp
from jax import lax
from jax.experimental import pallas as pl
from jax.experimental.pallas import tpu as pltpu
from jax.experimental.pallas import tpu_sc as plsc

_N, _E = 10000, 320000
_GD = 32
_EV = 20480
_E2 = _EV * 16
_NC, _NS = 2, 16
_EGA = 4
_CGA = 8
_RPA = _EV // _EGA
_RPD = _EV // 32
_CH = 512

_mesh = plsc.VectorSubcoreMesh(
    core_axis_name="c", subcore_axis_name="s", num_cores=_NC, num_subcores=_NS
)


_SLB = 4 * _N
_ECH = _CH * 16
_EPA = _E2 // _EGA
_EPD = _E2 // 32


@functools.partial(
    pl.kernel,
    out_type=jax.ShapeDtypeStruct((_EGA * _CGA * _SLB,), jnp.float32),
    mesh=_mesh,
    scratch_types=[
        [pltpu.VMEM((_N,), jnp.float32)] * 4,
        [pltpu.VMEM((_N,), jnp.float32)] * 4,
        pltpu.VMEM((_ECH,), jnp.int32),
        pltpu.VMEM((_ECH,), jnp.int32),
    ],
    compiler_params=pltpu.CompilerParams(needs_layout_passes=False),
)
def _sc_aggregate_body(gt_hbm, src_hbm, dst_hbm, zeros_hbm, out_hbm,
                       tbl_v, acc_v, src_v, dst_v):
    c = lax.axis_index("c")
    s = lax.axis_index("s")
    wid = s * _NC + c
    eg = wid // _CGA
    cg = wid % _CGA
    for t in range(4):
        pltpu.sync_copy(gt_hbm.at[pl.ds(cg * _SLB + t * _N, _N)], tbl_v[t])
        pltpu.sync_copy(zeros_hbm, acc_v[t])
    ebase = eg * _EPA

    zf = jnp.zeros((16,), jnp.float32)
    zi = jnp.zeros((16,), jnp.int32)

    @pl.loop(0, _EPA // _ECH, init_carry=(zf, zf, zf, zf, zi))
    def _chunks(ch, carry):
        pltpu.sync_copy(src_hbm.at[pl.ds(ebase + ch * _ECH, _ECH)], src_v)
        pltpu.sync_copy(dst_hbm.at[pl.ds(ebase + ch * _ECH, _ECH)], dst_v)

        @pl.loop(0, _CH, init_carry=carry, unroll=8)
        def _rows(r, cy):
            pv0, pv1, pv2, pv3, pdst = cy
            for t, pv in enumerate((pv0, pv1, pv2, pv3)):
                plsc.addupdate_scatter(acc_v[t], [pdst], pv)
            srcv = src_v[pl.ds(r * 16, 16)]
            dstv = dst_v[pl.ds(r * 16, 16)]
            m = srcv >= 0
            srcc = jnp.maximum(srcv, 0)
            dstc = jnp.maximum(dstv, 0)
            nv = []
            for t in range(4):
                g = plsc.load_gather(tbl_v[t], [srcc])
                nv.append(jnp.where(m, g, 0.0))
            return (nv[0], nv[1], nv[2], nv[3], dstc)

        return _rows

    fv0, fv1, fv2, fv3, fdst = _chunks
    for t, fv in enumerate((fv0, fv1, fv2, fv3)):
        plsc.addupdate_scatter(acc_v[t], [fdst], fv)

    for t in range(4):
        pltpu.sync_copy(acc_v[t], out_hbm.at[pl.ds(wid * _SLB + t * _N, _N)])


@functools.partial(
    pl.kernel,
    out_type=jax.ShapeDtypeStruct((32 * _N,), jnp.float32),
    mesh=_mesh,
    scratch_types=[
        pltpu.VMEM((_N,), jnp.float32),
        pltpu.VMEM((_EPD,), jnp.int32),
    ],
    compiler_params=pltpu.CompilerParams(needs_layout_passes=False),
)
def _sc_degree_body(dst_hbm, zeros_hbm, out_hbm, acc_v, dst_v):
    c = lax.axis_index("c")
    s = lax.axis_index("s")
    wid = s * _NC + c
    pltpu.sync_copy(zeros_hbm, acc_v)
    pltpu.sync_copy(dst_hbm.at[pl.ds(wid * _EPD, _EPD)], dst_v)

    @pl.loop(0, _EPD // 16, unroll=4)
    def _rows(r):
        dstv = dst_v[pl.ds(r * 16, 16)]
        m = dstv >= 0
        dstc = jnp.maximum(dstv, 0)
        one16 = jnp.full((16,), 1.0, jnp.float32)
        plsc.addupdate_scatter(acc_v, [dstc], one16, mask=m)

    pltpu.sync_copy(acc_v, out_hbm.at[pl.ds(wid * _N, _N)])


def _embed_body(x_ref, s_ref, r_ref, we1_ref, be1_ref, we2_ref, be2_ref,
                wg1_ref, z1_ref):
    sub = jnp.dot(s_ref[...], r_ref[...], preferred_element_type=jnp.float32)
    h = jnp.concatenate([x_ref[...], sub], axis=1)
    t = jnp.tanh(jnp.dot(h, we1_ref[...], preferred_element_type=jnp.float32)
                 + be1_ref[...])
    h2 = jnp.tanh(jnp.dot(t, we2_ref[...], preferred_element_type=jnp.float32)
                  + be2_ref[...])
    z1_ref[...] = jnp.dot(h2, wg1_ref[...],
                          preferred_element_type=jnp.float32).T


_embed_call = pl.pallas_call(
    _embed_body,
    out_shape=jax.ShapeDtypeStruct((_GD, _N), jnp.float32),
)


def _scale_body(d_ref, z1_ref, dinv_ref, g_ref):
    deg = jnp.sum(d_ref[...], axis=0, keepdims=True) + 1.0
    dinvT = lax.rsqrt(deg)
    dinv_ref[...] = dinvT
    g_ref[...] = dinvT * z1_ref[...]


_scale_call = pl.pallas_call(
    _scale_body,
    out_shape=[
        jax.ShapeDtypeStruct((1, _N), jnp.float32),
        jax.ShapeDtypeStruct((_GD, _N), jnp.float32),
    ],
)


def _agg_from_parts(p_ref):
    parts = []
    for b in range(_CGA):
        parts.append(p_ref[0, b] + p_ref[1, b] + p_ref[2, b] + p_ref[3, b])
    return jnp.concatenate(parts, axis=0)


def _layer_body(p_ref, g_ref, dinv_ref, b_ref, w_ref, gout_ref):
    dinvT = dinv_ref[...]
    aggT = _agg_from_parts(p_ref)
    hT = jnp.maximum(dinvT * (aggT + g_ref[...]) + b_ref[...], 0.0)
    gout_ref[...] = dinvT * jnp.dot(w_ref[...].T, hT,
                                    preferred_element_type=jnp.float32)


_layer_call = pl.pallas_call(
    _layer_body,
    out_shape=jax.ShapeDtypeStruct((_GD, _N), jnp.float32),
)


def _final_body(p_ref, g_ref, dinv_ref, b3_ref, wp1_ref, bp1_ref, wp2_ref,
                bp2_ref, pri_ref, out_ref):
    dinvT = dinv_ref[...]
    aggT = _agg_from_parts(p_ref)
    hT = jnp.maximum(dinvT * (aggT + g_ref[...]) + b3_ref[...], 0.0)
    h = hT.T
    t = jnp.tanh(jnp.dot(h, wp1_ref[...], preferred_element_type=jnp.float32)
                 + bp1_ref[...])
    o = jnp.tanh(jnp.dot(t, wp2_ref[...], preferred_element_type=jnp.float32)
                 + bp2_ref[...])
    out_ref[...] = o + pri_ref[...]


_final_call = pl.pallas_call(
    _final_body,
    out_shape=jax.ShapeDtypeStruct((_N, 16), jnp.float32),
)


def kernel(x, edge_index, priors, S, R, We1, be1, We2, be2,
           Wg1, bg1, Wg2, bg2, Wg3, bg3, Wp1, bp1, Wp2, bp2):
    f32, i32 = jnp.float32, jnp.int32
    pad = _E2 - _E
    srcp = jnp.concatenate([edge_index[0], jnp.full((pad,), -1, i32)])
    dstp = jnp.concatenate([edge_index[1], jnp.full((pad,), -1, i32)])
    zeros1 = jnp.zeros((_N,), f32)

    degp = _sc_degree_body(dstp, zeros1).reshape(32, _N)
    z1T = _embed_call(x, S, R, We1, be1.reshape(1, -1), We2,
                      be2.reshape(1, -1), Wg1)
    dinvT, g1T = _scale_call(degp, z1T)

    def _agg(gT):
        p = _sc_aggregate_body(gT.reshape(-1), srcp, dstp, zeros1)
        return p.reshape(_EGA, _CGA, 4, _N)

    p = _agg(g1T)
    g2T = _layer_call(p, g1T, dinvT, bg1.reshape(-1, 1), Wg2)
    p = _agg(g2T)
    g3T = _layer_call(p, g2T, dinvT, bg2.reshape(-1, 1), Wg3)
    p = _agg(g3T)
    out = _final_call(p, g3T, dinvT, bg3.reshape(-1, 1), Wp1,
                      bp1.reshape(1, -1), Wp2, bp2.reshape(1, -1), priors)
    return out
